# baseline (speedup 1.0000x reference)
import functools

import jax
import jax.numpy as jnp
from jax import lax
from jax.experimental import pallas as pl
from jax.experimental.pallas import tpu as pltpu
from jax.experimental.pallas import tpu_sc as plsc

VOCAB = 1000000
EMBED = 50
CTX = 5
HIDDEN = 256
NUM_CLASSES = 64
BATCH = 16384

NC = 2
NS = 16
NW = NC * NS

N_IDX = BATCH * CTX
PER_W = N_IDX // NW
CHUNK = 128
N_CHUNKS = PER_W // CHUNK
EPAD = 56

_sc_mesh = plsc.VectorSubcoreMesh(core_axis_name="c", subcore_axis_name="s")


@functools.partial(
    pl.kernel,
    mesh=_sc_mesh,
    out_type=jax.ShapeDtypeStruct((CTX, EMBED, BATCH), jnp.float32),
    scratch_types=[
        pltpu.VMEM((EPAD, CHUNK), jnp.int32),
        pltpu.VMEM((EMBED, CHUNK), jnp.float32),
        pltpu.SemaphoreType.DMA,
    ],
)
def _sc_gather(widx_hbm, tab_hbm, out_hbm, widx_v, dst_v, gsem):
    wid = lax.axis_index("s") * NC + lax.axis_index("c")

    @pl.loop(0, N_CHUNKS)
    def _chunk(j):
        chunk_id = wid * N_CHUNKS + j
        c = chunk_id * CHUNK // BATCH
        b0 = chunk_id * CHUNK % BATCH
        pltpu.sync_copy(widx_hbm.at[pl.ds(chunk_id * EPAD, EPAD)], widx_v)
        copies = [
            pltpu.async_copy(tab_hbm.at[widx_v.at[f]], dst_v.at[f], gsem)
            for f in range(EMBED)
        ]
        for cp in copies:
            cp.wait()
        pltpu.sync_copy(dst_v, out_hbm.at[c, :, pl.ds(b0, CHUNK)])


_BB = 2048


def _mlp_body(e_ref, w1_ref, b1_ref, w2_ref, b2_ref, out_ref):
    acc = jnp.broadcast_to(b1_ref[...], (_BB, HIDDEN))
    for c in range(CTX):
        acc = acc + lax.dot_general(
            e_ref[c], w1_ref[c],
            dimension_numbers=(((0,), (0,)), ((), ())),
            preferred_element_type=jnp.float32,
            precision=lax.Precision.HIGHEST)
    h = jnp.tanh(acc)
    logits = jnp.dot(h, w2_ref[...],
                     preferred_element_type=jnp.float32,
                     precision=lax.Precision.HIGHEST) + b2_ref[...]
    m = jnp.max(logits, axis=1, keepdims=True)
    l = logits - m
    lse = jnp.log(jnp.sum(jnp.exp(l), axis=1, keepdims=True))
    out_ref[...] = l - lse


def _mlp(e, W1c, b1, W2, b2):
    grid = (BATCH // _BB,)
    return pl.pallas_call(
        _mlp_body,
        grid=grid,
        in_specs=[
            pl.BlockSpec((CTX, EMBED, _BB), lambda i: (0, 0, i)),
            pl.BlockSpec((CTX, EMBED, HIDDEN), lambda i: (0, 0, 0)),
            pl.BlockSpec((1, HIDDEN), lambda i: (0, 0)),
            pl.BlockSpec((HIDDEN, NUM_CLASSES), lambda i: (0, 0)),
            pl.BlockSpec((1, NUM_CLASSES), lambda i: (0, 0)),
        ],
        out_specs=pl.BlockSpec((_BB, NUM_CLASSES), lambda i: (i, 0)),
        out_shape=jax.ShapeDtypeStruct((BATCH, NUM_CLASSES), jnp.float32),
    )(e, W1c, b1, W2, b2)


def kernel(x, table, W1, b1, W2, b2):
    tab_flat = table.T.reshape(-1)
    r = x.T.astype(jnp.int32).reshape(NW * N_CHUNKS, 1, CHUNK)
    f = jnp.where(jnp.arange(EPAD) < EMBED,
                  jnp.arange(EPAD) * VOCAB, 0).astype(jnp.int32)
    widx = (r + f.reshape(1, EPAD, 1)).reshape(NW * N_CHUNKS * EPAD, CHUNK)
    e = _sc_gather(widx, tab_flat)
    W1c = W1.reshape(CTX, EMBED, HIDDEN)
    return _mlp(e, W1c, b1.reshape(1, HIDDEN), W2, b2.reshape(1, NUM_CLASSES))


# speedup vs baseline: 1.4022x; 1.4022x over previous
import functools

import jax
import jax.numpy as jnp
from jax import lax
from jax.experimental import pallas as pl
from jax.experimental.pallas import tpu as pltpu
from jax.experimental.pallas import tpu_sc as plsc

VOCAB = 1000000
EMBED = 50
CTX = 5
HIDDEN = 256
NUM_CLASSES = 64
BATCH = 16384

NC = 2
NS = 16
NW = NC * NS

N_IDX = BATCH * CTX
PER_W = N_IDX // NW
CHUNK = 128
N_CHUNKS = PER_W // CHUNK
EPAD = 56

_sc_mesh = plsc.VectorSubcoreMesh(core_axis_name="c", subcore_axis_name="s")

VB = 512
V_MAIN = (VOCAB // 128) * 128
NVB = V_MAIN // VB
NF8 = (EMBED + 7) // 8
F_LAST = EMBED - 8 * (NF8 - 1)
TAIL = VOCAB - V_MAIN
NF8_FULL = EMBED // 8
N_MAIN = NF8_FULL * NVB
N_UNITS = N_MAIN + NVB + 1
UNITS_PW = -(-N_UNITS // NW)


@functools.partial(
    pl.kernel,
    mesh=_sc_mesh,
    out_type=jax.ShapeDtypeStruct((EMBED * VOCAB,), jnp.float32),
    scratch_types=[
        pltpu.VMEM((8, VB), jnp.float32),
        pltpu.VMEM((2, VB), jnp.float32),
        pltpu.VMEM((EMBED, TAIL), jnp.float32),
        pltpu.SemaphoreType.DMA,
    ],
)
def _sc_detile(tabT_hbm, last2_hbm, tail_hbm, out_hbm, blk_v, blk2_v, tail_v, wsem):
    wid = lax.axis_index("s") * NC + lax.axis_index("c")

    @pl.loop(0, UNITS_PW)
    def _unit(i):
        unit = wid + i * NW

        @pl.when(unit < N_MAIN)
        def _full():
            f8 = unit // NVB
            v0 = pl.multiple_of((unit % NVB) * VB, VB)
            pltpu.sync_copy(
                tabT_hbm.at[pl.ds(pl.multiple_of(f8 * 8, 8), 8), pl.ds(v0, VB)],
                blk_v)
            copies = []
            for s in range(8):
                for k in range(VB // 128):
                    copies.append(pltpu.async_copy(
                        blk_v.at[s, pl.ds(k * 128, 128)],
                        out_hbm.at[pl.ds((f8 * 8 + s) * VOCAB + v0 + k * 128,
                                         128)],
                        wsem))
            for cp in copies:
                cp.wait()

        @pl.when(jnp.logical_and(unit >= N_MAIN, unit < N_MAIN + NVB))
        def _last2():
            v0 = pl.multiple_of((unit - N_MAIN) * VB, VB)
            pltpu.sync_copy(last2_hbm.at[:, pl.ds(v0, VB)], blk2_v)
            copies = []
            for s in range(2):
                for k in range(VB // 128):
                    copies.append(pltpu.async_copy(
                        blk2_v.at[s, pl.ds(k * 128, 128)],
                        out_hbm.at[pl.ds((48 + s) * VOCAB + v0 + k * 128, 128)],
                        wsem))
            for cp in copies:
                cp.wait()

        @pl.when(unit == N_MAIN + NVB)
        def _tail():
            pltpu.sync_copy(tail_hbm, tail_v)
            copies = [
                pltpu.async_copy(
                    tail_v.at[f],
                    out_hbm.at[pl.ds(f * VOCAB + V_MAIN, TAIL)], wsem)
                for f in range(EMBED)
            ]
            for cp in copies:
                cp.wait()


@functools.partial(
    pl.kernel,
    mesh=_sc_mesh,
    out_type=jax.ShapeDtypeStruct((CTX, EMBED, BATCH), jnp.float32),
    scratch_types=[
        pltpu.VMEM((EPAD, CHUNK), jnp.int32),
        pltpu.VMEM((EMBED, CHUNK), jnp.float32),
        pltpu.SemaphoreType.DMA,
    ],
)
def _sc_gather(widx_hbm, tab_hbm, out_hbm, widx_v, dst_v, gsem):
    wid = lax.axis_index("s") * NC + lax.axis_index("c")

    @pl.loop(0, N_CHUNKS)
    def _chunk(j):
        chunk_id = wid * N_CHUNKS + j
        c = chunk_id * CHUNK // BATCH
        b0 = chunk_id * CHUNK % BATCH
        pltpu.sync_copy(widx_hbm.at[pl.ds(chunk_id * EPAD, EPAD)], widx_v)
        copies = [
            pltpu.async_copy(tab_hbm.at[widx_v.at[f]], dst_v.at[f], gsem)
            for f in range(EMBED)
        ]
        for cp in copies:
            cp.wait()
        pltpu.sync_copy(dst_v, out_hbm.at[c, :, pl.ds(b0, CHUNK)])


_BB = 2048


def _mlp_body(e_ref, w1_ref, b1_ref, w2_ref, b2_ref, out_ref):
    acc = jnp.broadcast_to(b1_ref[...], (_BB, HIDDEN))
    for c in range(CTX):
        acc = acc + lax.dot_general(
            e_ref[c], w1_ref[c],
            dimension_numbers=(((0,), (0,)), ((), ())),
            preferred_element_type=jnp.float32,
            precision=lax.Precision.HIGHEST)
    h = jnp.tanh(acc)
    logits = jnp.dot(h, w2_ref[...],
                     preferred_element_type=jnp.float32,
                     precision=lax.Precision.HIGHEST) + b2_ref[...]
    m = jnp.max(logits, axis=1, keepdims=True)
    l = logits - m
    lse = jnp.log(jnp.sum(jnp.exp(l), axis=1, keepdims=True))
    out_ref[...] = l - lse


def _mlp(e, W1c, b1, W2, b2):
    grid = (BATCH // _BB,)
    return pl.pallas_call(
        _mlp_body,
        grid=grid,
        in_specs=[
            pl.BlockSpec((CTX, EMBED, _BB), lambda i: (0, 0, i)),
            pl.BlockSpec((CTX, EMBED, HIDDEN), lambda i: (0, 0, 0)),
            pl.BlockSpec((1, HIDDEN), lambda i: (0, 0)),
            pl.BlockSpec((HIDDEN, NUM_CLASSES), lambda i: (0, 0)),
            pl.BlockSpec((1, NUM_CLASSES), lambda i: (0, 0)),
        ],
        out_specs=pl.BlockSpec((_BB, NUM_CLASSES), lambda i: (i, 0)),
        out_shape=jax.ShapeDtypeStruct((BATCH, NUM_CLASSES), jnp.float32),
    )(e, W1c, b1, W2, b2)


def kernel(x, table, W1, b1, W2, b2):
    tab_flat = jnp.concatenate([table[:, f] for f in range(EMBED)])
    r = x.T.astype(jnp.int32).reshape(NW * N_CHUNKS, 1, CHUNK)
    f = jnp.where(jnp.arange(EPAD) < EMBED,
                  jnp.arange(EPAD) * VOCAB, 0).astype(jnp.int32)
    widx = (r + f.reshape(1, EPAD, 1)).reshape(NW * N_CHUNKS * EPAD, CHUNK)
    e = _sc_gather(widx, tab_flat)
    W1c = W1.reshape(CTX, EMBED, HIDDEN)
    return _mlp(e, W1c, b1.reshape(1, HIDDEN), W2, b2.reshape(1, NUM_CLASSES))


# speedup vs baseline: 4.9820x; 3.5529x over previous
import functools

import jax
import jax.numpy as jnp
from jax import lax
from jax.experimental import pallas as pl
from jax.experimental.pallas import tpu as pltpu
from jax.experimental.pallas import tpu_sc as plsc

VOCAB = 1000000
EMBED = 50
CTX = 5
HIDDEN = 256
NUM_CLASSES = 64
BATCH = 16384

NC = 2
NS = 16
NW = NC * NS

N_IDX = BATCH * CTX
PER_W = N_IDX // NW
CHUNK = 128
N_CHUNKS = PER_W // CHUNK
EPAD = 56

_sc_mesh = plsc.VectorSubcoreMesh(core_axis_name="c", subcore_axis_name="s")

VB = 512
V_MAIN = (VOCAB // 128) * 128
NVB = V_MAIN // VB
NF8 = (EMBED + 7) // 8
F_LAST = EMBED - 8 * (NF8 - 1)
TAIL = VOCAB - V_MAIN
NF8_FULL = EMBED // 8
N_MAIN = NF8_FULL * NVB
N_UNITS = N_MAIN + NVB + 1
UNITS_PW = -(-N_UNITS // NW)


@functools.partial(
    pl.kernel,
    mesh=_sc_mesh,
    out_type=jax.ShapeDtypeStruct((EMBED * VOCAB,), jnp.float32),
    scratch_types=[
        pltpu.VMEM((8, VB), jnp.float32),
        pltpu.VMEM((8 * VB,), jnp.float32),
        pltpu.VMEM((2, VB), jnp.float32),
        pltpu.VMEM((EMBED, TAIL), jnp.float32),
        pltpu.VMEM((EMBED * TAIL,), jnp.float32),
        pltpu.SemaphoreType.DMA,
    ],
)
def _sc_detile(tabT_hbm, last2_hbm, tail_hbm, out_hbm,
               blk_v, row_v, blk2_v, tail_v, trow_v, wsem):
    wid = lax.axis_index("s") * NC + lax.axis_index("c")

    def _compact_and_write(src_v, nrows, width, base_f, v0, chunk):
        for s in range(nrows):
            for t in range(width // 16):
                row_v[pl.ds(s * width + t * 16, 16)] = (
                    src_v[s, pl.ds(t * 16, 16)])
        copies = [
            pltpu.async_copy(
                row_v.at[pl.ds(s * width, width)],
                out_hbm.at[pl.ds((base_f + s) * VOCAB + v0, width)], wsem)
            for s in range(nrows)
        ]
        for cp in copies:
            cp.wait()

    @pl.loop(0, UNITS_PW)
    def _unit(i):
        unit = wid + i * NW

        @pl.when(unit < N_MAIN)
        def _full():
            f8 = unit // NVB
            v0 = pl.multiple_of((unit % NVB) * VB, VB)
            pltpu.sync_copy(
                tabT_hbm.at[pl.ds(pl.multiple_of(f8 * 8, 8), 8), pl.ds(v0, VB)],
                blk_v)
            _compact_and_write(blk_v, 8, VB, f8 * 8, v0, unit)

        @pl.when(jnp.logical_and(unit >= N_MAIN, unit < N_MAIN + NVB))
        def _last2():
            v0 = pl.multiple_of((unit - N_MAIN) * VB, VB)
            pltpu.sync_copy(last2_hbm.at[:, pl.ds(v0, VB)], blk2_v)
            _compact_and_write(blk2_v, 2, VB, 48, v0, unit)

        @pl.when(unit == N_MAIN + NVB)
        def _tail():
            pltpu.sync_copy(tail_hbm, tail_v)
            for f in range(EMBED):
                for t in range(TAIL // 16):
                    trow_v[pl.ds(f * TAIL + t * 16, 16)] = (
                        tail_v[f, pl.ds(t * 16, 16)])
            copies = [
                pltpu.async_copy(
                    trow_v.at[pl.ds(f * TAIL, TAIL)],
                    out_hbm.at[pl.ds(f * VOCAB + V_MAIN, TAIL)], wsem)
                for f in range(EMBED)
            ]
            for cp in copies:
                cp.wait()


@functools.partial(
    pl.kernel,
    mesh=_sc_mesh,
    out_type=jax.ShapeDtypeStruct((CTX, EMBED, BATCH), jnp.float32),
    scratch_types=[
        pltpu.VMEM((EPAD, CHUNK), jnp.int32),
        pltpu.VMEM((EMBED, CHUNK), jnp.float32),
        pltpu.SemaphoreType.DMA,
    ],
)
def _sc_gather(widx_hbm, tab_hbm, out_hbm, widx_v, dst_v, gsem):
    wid = lax.axis_index("s") * NC + lax.axis_index("c")

    @pl.loop(0, N_CHUNKS)
    def _chunk(j):
        chunk_id = wid * N_CHUNKS + j
        c = chunk_id * CHUNK // BATCH
        b0 = chunk_id * CHUNK % BATCH
        pltpu.sync_copy(widx_hbm.at[pl.ds(chunk_id * EPAD, EPAD)], widx_v)
        copies = [
            pltpu.async_copy(tab_hbm.at[widx_v.at[f]], dst_v.at[f], gsem)
            for f in range(EMBED)
        ]
        for cp in copies:
            cp.wait()
        pltpu.sync_copy(dst_v, out_hbm.at[c, :, pl.ds(b0, CHUNK)])


_BB = 2048


def _mlp_body(e_ref, w1_ref, b1_ref, w2_ref, b2_ref, out_ref):
    acc = jnp.broadcast_to(b1_ref[...], (_BB, HIDDEN))
    for c in range(CTX):
        acc = acc + lax.dot_general(
            e_ref[c], w1_ref[c],
            dimension_numbers=(((0,), (0,)), ((), ())),
            preferred_element_type=jnp.float32,
            precision=lax.Precision.HIGHEST)
    h = jnp.tanh(acc)
    logits = jnp.dot(h, w2_ref[...],
                     preferred_element_type=jnp.float32,
                     precision=lax.Precision.HIGHEST) + b2_ref[...]
    m = jnp.max(logits, axis=1, keepdims=True)
    l = logits - m
    lse = jnp.log(jnp.sum(jnp.exp(l), axis=1, keepdims=True))
    out_ref[...] = l - lse


def _mlp(e, W1c, b1, W2, b2):
    grid = (BATCH // _BB,)
    return pl.pallas_call(
        _mlp_body,
        grid=grid,
        in_specs=[
            pl.BlockSpec((CTX, EMBED, _BB), lambda i: (0, 0, i)),
            pl.BlockSpec((CTX, EMBED, HIDDEN), lambda i: (0, 0, 0)),
            pl.BlockSpec((1, HIDDEN), lambda i: (0, 0)),
            pl.BlockSpec((HIDDEN, NUM_CLASSES), lambda i: (0, 0)),
            pl.BlockSpec((1, NUM_CLASSES), lambda i: (0, 0)),
        ],
        out_specs=pl.BlockSpec((_BB, NUM_CLASSES), lambda i: (i, 0)),
        out_shape=jax.ShapeDtypeStruct((BATCH, NUM_CLASSES), jnp.float32),
    )(e, W1c, b1, W2, b2)


def kernel(x, table, W1, b1, W2, b2):
    tabT = table.T
    tab_flat = _sc_detile(tabT, tabT[48:50, :], tabT[:, V_MAIN:])
    r = x.T.astype(jnp.int32).reshape(NW * N_CHUNKS, 1, CHUNK)
    f = jnp.where(jnp.arange(EPAD) < EMBED,
                  jnp.arange(EPAD) * VOCAB, 0).astype(jnp.int32)
    widx = (r + f.reshape(1, EPAD, 1)).reshape(NW * N_CHUNKS * EPAD, CHUNK)
    e = _sc_gather(widx, tab_flat)
    W1c = W1.reshape(CTX, EMBED, HIDDEN)
    return _mlp(e, W1c, b1.reshape(1, HIDDEN), W2, b2.reshape(1, NUM_CLASSES))


# speedup vs baseline: 6.5023x; 1.3052x over previous
import functools

import jax
import jax.numpy as jnp
from jax import lax
from jax.experimental import pallas as pl
from jax.experimental.pallas import tpu as pltpu
from jax.experimental.pallas import tpu_sc as plsc

VOCAB = 1000000
EMBED = 50
CTX = 5
HIDDEN = 256
NUM_CLASSES = 64
BATCH = 16384

NC = 2
NS = 16
NW = NC * NS

N_IDX = BATCH * CTX
PER_W = N_IDX // NW
CHUNK = 128
N_CHUNKS = PER_W // CHUNK
EPAD = 56

_sc_mesh = plsc.VectorSubcoreMesh(core_axis_name="c", subcore_axis_name="s")

VB = 1536
V_MAIN = (VOCAB // 128) * 128
NVB = V_MAIN // VB
TAIL = VOCAB - V_MAIN
NF8_FULL = EMBED // 8
N_MAIN = NF8_FULL * NVB
N_REST = NVB + 1
MAIN_PW = -(-N_MAIN // NW)
PAIRS_PW = -(-MAIN_PW // 2)
REST_PW = -(-N_REST // NW)


@functools.partial(
    pl.kernel,
    mesh=_sc_mesh,
    out_type=jax.ShapeDtypeStruct((EMBED * VOCAB,), jnp.float32),
    scratch_types=[
        pltpu.VMEM((8, VB), jnp.float32),
        pltpu.VMEM((8, VB), jnp.float32),
        pltpu.VMEM((8 * VB,), jnp.float32),
        pltpu.VMEM((8 * VB,), jnp.float32),
        pltpu.VMEM((2, VB), jnp.float32),
        pltpu.VMEM((EMBED, TAIL), jnp.float32),
        pltpu.VMEM((EMBED * TAIL,), jnp.float32),
        pltpu.SemaphoreType.DMA,
        pltpu.SemaphoreType.DMA,
        pltpu.SemaphoreType.DMA,
    ],
)
def _sc_detile(tabT_hbm, last2_hbm, tail_hbm, out_hbm,
               blk_a, blk_b, row_a, row_b, blk2_v, tail_v, trow_v,
               rsem_a, rsem_b, wsem):
    wid = lax.axis_index("s") * NC + lax.axis_index("c")

    def _src(unit):
        f8 = unit // NVB
        v0 = pl.multiple_of((unit % NVB) * VB, VB)
        return tabT_hbm.at[pl.ds(pl.multiple_of(f8 * 8, 8), 8),
                           pl.ds(v0, VB)], f8 * 8, v0

    def _compact_and_write(src_v, row_v, nrows, width, base_f, v0):
        for s in range(nrows):
            for t in range(width // 16):
                row_v[pl.ds(s * width + t * 16, 16)] = (
                    src_v[s, pl.ds(t * 16, 16)])
        copies = [
            pltpu.async_copy(
                row_v.at[pl.ds(s * width, width)],
                out_hbm.at[pl.ds((base_f + s) * VOCAB + v0, width)], wsem)
            for s in range(nrows)
        ]
        for cp in copies:
            cp.wait()

    @pl.loop(0, PAIRS_PW)
    def _pair(p):
        u_a = wid + (2 * p) * NW
        u_b = wid + (2 * p + 1) * NW

        @pl.when(u_a < N_MAIN)
        def _fire_a():
            src, _, _ = _src(u_a)
            pltpu.async_copy(src, blk_a, rsem_a)

        @pl.when(u_b < N_MAIN)
        def _fire_b():
            src, _, _ = _src(u_b)
            pltpu.async_copy(src, blk_b, rsem_b)

        @pl.when(u_a < N_MAIN)
        def _do_a():
            src, base_f, v0 = _src(u_a)
            pltpu.make_async_copy(src, blk_a, rsem_a).wait()
            _compact_and_write(blk_a, row_a, 8, VB, base_f, v0)

        @pl.when(u_b < N_MAIN)
        def _do_b():
            src, base_f, v0 = _src(u_b)
            pltpu.make_async_copy(src, blk_b, rsem_b).wait()
            _compact_and_write(blk_b, row_b, 8, VB, base_f, v0)

    @pl.loop(0, REST_PW)
    def _rest(i):
        unit = wid + i * NW

        @pl.when(unit < NVB)
        def _last2():
            v0 = pl.multiple_of(unit * VB, VB)
            pltpu.sync_copy(last2_hbm.at[:, pl.ds(v0, VB)], blk2_v)
            _compact_and_write(blk2_v, row_a, 2, VB, 48, v0)

        @pl.when(unit == NVB)
        def _tail():
            pltpu.sync_copy(tail_hbm, tail_v)
            for f in range(EMBED):
                for t in range(TAIL // 16):
                    trow_v[pl.ds(f * TAIL + t * 16, 16)] = (
                        tail_v[f, pl.ds(t * 16, 16)])
            copies = [
                pltpu.async_copy(
                    trow_v.at[pl.ds(f * TAIL, TAIL)],
                    out_hbm.at[pl.ds(f * VOCAB + V_MAIN, TAIL)], wsem)
                for f in range(EMBED)
            ]
            for cp in copies:
                cp.wait()


@functools.partial(
    pl.kernel,
    mesh=_sc_mesh,
    out_type=jax.ShapeDtypeStruct((CTX, EMBED, BATCH), jnp.float32),
    scratch_types=[
        pltpu.VMEM((EPAD, CHUNK), jnp.int32),
        pltpu.VMEM((EMBED, CHUNK), jnp.float32),
        pltpu.SemaphoreType.DMA,
    ],
)
def _sc_gather(widx_hbm, tab_hbm, out_hbm, widx_v, dst_v, gsem):
    wid = lax.axis_index("s") * NC + lax.axis_index("c")

    @pl.loop(0, N_CHUNKS)
    def _chunk(j):
        chunk_id = wid * N_CHUNKS + j
        c = chunk_id * CHUNK // BATCH
        b0 = chunk_id * CHUNK % BATCH
        pltpu.sync_copy(widx_hbm.at[pl.ds(chunk_id * EPAD, EPAD)], widx_v)
        copies = [
            pltpu.async_copy(tab_hbm.at[widx_v.at[f]], dst_v.at[f], gsem)
            for f in range(EMBED)
        ]
        for cp in copies:
            cp.wait()
        pltpu.sync_copy(dst_v, out_hbm.at[c, :, pl.ds(b0, CHUNK)])


_BB = 2048


def _mlp_body(e_ref, w1_ref, b1_ref, w2_ref, b2_ref, out_ref):
    acc = jnp.broadcast_to(b1_ref[...], (_BB, HIDDEN))
    for c in range(CTX):
        acc = acc + lax.dot_general(
            e_ref[c], w1_ref[c],
            dimension_numbers=(((0,), (0,)), ((), ())),
            preferred_element_type=jnp.float32,
            precision=lax.Precision.HIGHEST)
    h = jnp.tanh(acc)
    logits = jnp.dot(h, w2_ref[...],
                     preferred_element_type=jnp.float32,
                     precision=lax.Precision.HIGHEST) + b2_ref[...]
    m = jnp.max(logits, axis=1, keepdims=True)
    l = logits - m
    lse = jnp.log(jnp.sum(jnp.exp(l), axis=1, keepdims=True))
    out_ref[...] = l - lse


def _mlp(e, W1c, b1, W2, b2):
    grid = (BATCH // _BB,)
    return pl.pallas_call(
        _mlp_body,
        grid=grid,
        in_specs=[
            pl.BlockSpec((CTX, EMBED, _BB), lambda i: (0, 0, i)),
            pl.BlockSpec((CTX, EMBED, HIDDEN), lambda i: (0, 0, 0)),
            pl.BlockSpec((1, HIDDEN), lambda i: (0, 0)),
            pl.BlockSpec((HIDDEN, NUM_CLASSES), lambda i: (0, 0)),
            pl.BlockSpec((1, NUM_CLASSES), lambda i: (0, 0)),
        ],
        out_specs=pl.BlockSpec((_BB, NUM_CLASSES), lambda i: (i, 0)),
        out_shape=jax.ShapeDtypeStruct((BATCH, NUM_CLASSES), jnp.float32),
    )(e, W1c, b1, W2, b2)


def kernel(x, table, W1, b1, W2, b2):
    tabT = table.T
    tab_flat = _sc_detile(tabT, tabT[48:50, :], tabT[:, V_MAIN:])
    r = x.T.astype(jnp.int32).reshape(NW * N_CHUNKS, 1, CHUNK)
    f = jnp.where(jnp.arange(EPAD) < EMBED,
                  jnp.arange(EPAD) * VOCAB, 0).astype(jnp.int32)
    widx = (r + f.reshape(1, EPAD, 1)).reshape(NW * N_CHUNKS * EPAD, CHUNK)
    e = _sc_gather(widx, tab_flat)
    W1c = W1.reshape(CTX, EMBED, HIDDEN)
    return _mlp(e, W1c, b1.reshape(1, HIDDEN), W2, b2.reshape(1, NUM_CLASSES))


# speedup vs baseline: 6.8017x; 1.0460x over previous
import functools

import jax
import jax.numpy as jnp
from jax import lax
from jax.experimental import pallas as pl
from jax.experimental.pallas import tpu as pltpu
from jax.experimental.pallas import tpu_sc as plsc

VOCAB = 1000000
EMBED = 50
CTX = 5
HIDDEN = 256
NUM_CLASSES = 64
BATCH = 16384

NC = 2
NS = 16
NW = NC * NS

N_IDX = BATCH * CTX
PER_W = N_IDX // NW
CHUNK = 128
N_CHUNKS = PER_W // CHUNK
EPAD = 56

_sc_mesh = plsc.VectorSubcoreMesh(core_axis_name="c", subcore_axis_name="s")

VB = 1536
V_MAIN = (VOCAB // 128) * 128
NVB = V_MAIN // VB
TAIL = VOCAB - V_MAIN
NF8_FULL = EMBED // 8
N_MAIN = NF8_FULL * NVB
N_REST = NVB + 1
MAIN_PW = -(-N_MAIN // NW)
PAIRS_PW = -(-MAIN_PW // 2)
REST_PW = -(-N_REST // NW)


@functools.partial(
    pl.kernel,
    mesh=_sc_mesh,
    out_type=jax.ShapeDtypeStruct((EMBED * VOCAB,), jnp.float32),
    scratch_types=[
        pltpu.VMEM((8, VB), jnp.float32),
        pltpu.VMEM((8, VB), jnp.float32),
        pltpu.VMEM((8 * VB,), jnp.float32),
        pltpu.VMEM((8 * VB,), jnp.float32),
        pltpu.VMEM((2, VB), jnp.float32),
        pltpu.VMEM((EMBED, TAIL), jnp.float32),
        pltpu.VMEM((EMBED * TAIL,), jnp.float32),
        pltpu.SemaphoreType.DMA,
        pltpu.SemaphoreType.DMA,
        pltpu.SemaphoreType.DMA,
    ],
)
def _sc_detile(tabT_hbm, last2_hbm, tail_hbm, out_hbm,
               blk_a, blk_b, row_a, row_b, blk2_v, tail_v, trow_v,
               rsem_a, rsem_b, wsem):
    wid = lax.axis_index("s") * NC + lax.axis_index("c")

    def _src(unit):
        f8 = unit // NVB
        v0 = pl.multiple_of((unit % NVB) * VB, VB)
        return tabT_hbm.at[pl.ds(pl.multiple_of(f8 * 8, 8), 8),
                           pl.ds(v0, VB)], f8 * 8, v0

    def _compact_and_write(src_v, row_v, nrows, width, base_f, v0):
        for s in range(nrows):
            for t in range(width // 16):
                row_v[pl.ds(s * width + t * 16, 16)] = (
                    src_v[s, pl.ds(t * 16, 16)])
        copies = [
            pltpu.async_copy(
                row_v.at[pl.ds(s * width, width)],
                out_hbm.at[pl.ds((base_f + s) * VOCAB + v0, width)], wsem)
            for s in range(nrows)
        ]
        for cp in copies:
            cp.wait()

    @pl.loop(0, PAIRS_PW)
    def _pair(p):
        u_a = wid + (2 * p) * NW
        u_b = wid + (2 * p + 1) * NW

        @pl.when(u_a < N_MAIN)
        def _fire_a():
            src, _, _ = _src(u_a)
            pltpu.async_copy(src, blk_a, rsem_a)

        @pl.when(u_b < N_MAIN)
        def _fire_b():
            src, _, _ = _src(u_b)
            pltpu.async_copy(src, blk_b, rsem_b)

        @pl.when(u_a < N_MAIN)
        def _do_a():
            src, base_f, v0 = _src(u_a)
            pltpu.make_async_copy(src, blk_a, rsem_a).wait()
            _compact_and_write(blk_a, row_a, 8, VB, base_f, v0)

        @pl.when(u_b < N_MAIN)
        def _do_b():
            src, base_f, v0 = _src(u_b)
            pltpu.make_async_copy(src, blk_b, rsem_b).wait()
            _compact_and_write(blk_b, row_b, 8, VB, base_f, v0)

    @pl.loop(0, REST_PW)
    def _rest(i):
        unit = wid + i * NW

        @pl.when(unit < NVB)
        def _last2():
            v0 = pl.multiple_of(unit * VB, VB)
            pltpu.sync_copy(last2_hbm.at[:, pl.ds(v0, VB)], blk2_v)
            _compact_and_write(blk2_v, row_a, 2, VB, 48, v0)

        @pl.when(unit == NVB)
        def _tail():
            pltpu.sync_copy(tail_hbm, tail_v)
            for f in range(EMBED):
                for t in range(TAIL // 16):
                    trow_v[pl.ds(f * TAIL + t * 16, 16)] = (
                        tail_v[f, pl.ds(t * 16, 16)])
            copies = [
                pltpu.async_copy(
                    trow_v.at[pl.ds(f * TAIL, TAIL)],
                    out_hbm.at[pl.ds(f * VOCAB + V_MAIN, TAIL)], wsem)
                for f in range(EMBED)
            ]
            for cp in copies:
                cp.wait()


@functools.partial(
    pl.kernel,
    mesh=_sc_mesh,
    out_type=jax.ShapeDtypeStruct((CTX, EMBED, BATCH), jnp.float32),
    scratch_types=[
        pltpu.VMEM((CHUNK,), jnp.int32),
        pltpu.VMEM((CHUNK,), jnp.int32),
        pltpu.VMEM((EMBED, CHUNK), jnp.int32),
        pltpu.VMEM((EMBED, CHUNK), jnp.int32),
        pltpu.VMEM((EMBED, CHUNK), jnp.float32),
        pltpu.VMEM((EMBED, CHUNK), jnp.float32),
        pltpu.SemaphoreType.DMA,
        pltpu.SemaphoreType.DMA,
    ],
)
def _sc_gather(xr_hbm, tab_hbm, out_hbm,
               x_a, x_b, widx_a, widx_b, dst_a, dst_b, sem_a, sem_b):
    wid = lax.axis_index("s") * NC + lax.axis_index("c")

    def _stage_and_fire(chunk_id, x_v, widx_v, dst_v, sem):
        pltpu.sync_copy(xr_hbm.at[pl.ds(chunk_id * CHUNK, CHUNK)], x_v)
        for g in range(CHUNK // 16):
            rv = x_v[pl.ds(g * 16, 16)]
            for f in range(EMBED):
                widx_v[f, pl.ds(g * 16, 16)] = rv + (f * VOCAB)
        return [
            pltpu.async_copy(tab_hbm.at[widx_v.at[f]], dst_v.at[f], sem)
            for f in range(EMBED)
        ]

    def _drain_and_write(chunk_id, copies, dst_v):
        for cp in copies:
            cp.wait()
        c = chunk_id * CHUNK // BATCH
        b0 = chunk_id * CHUNK % BATCH
        pltpu.sync_copy(dst_v, out_hbm.at[c, :, pl.ds(b0, CHUNK)])

    @pl.loop(0, N_CHUNKS // 2)
    def _pair(p):
        u_a = wid * N_CHUNKS + 2 * p
        u_b = u_a + 1
        cp_a = _stage_and_fire(u_a, x_a, widx_a, dst_a, sem_a)
        cp_b = _stage_and_fire(u_b, x_b, widx_b, dst_b, sem_b)
        _drain_and_write(u_a, cp_a, dst_a)
        _drain_and_write(u_b, cp_b, dst_b)


_BB = 2048


def _mlp_body(e_ref, w1_ref, b1_ref, w2_ref, b2_ref, out_ref):
    acc = jnp.broadcast_to(b1_ref[...], (_BB, HIDDEN))
    for c in range(CTX):
        acc = acc + lax.dot_general(
            e_ref[c], w1_ref[c],
            dimension_numbers=(((0,), (0,)), ((), ())),
            preferred_element_type=jnp.float32,
            precision=lax.Precision.HIGHEST)
    h = jnp.tanh(acc)
    logits = jnp.dot(h, w2_ref[...],
                     preferred_element_type=jnp.float32,
                     precision=lax.Precision.HIGHEST) + b2_ref[...]
    m = jnp.max(logits, axis=1, keepdims=True)
    l = logits - m
    lse = jnp.log(jnp.sum(jnp.exp(l), axis=1, keepdims=True))
    out_ref[...] = l - lse


def _mlp(e, W1c, b1, W2, b2):
    grid = (BATCH // _BB,)
    return pl.pallas_call(
        _mlp_body,
        grid=grid,
        in_specs=[
            pl.BlockSpec((CTX, EMBED, _BB), lambda i: (0, 0, i)),
            pl.BlockSpec((CTX, EMBED, HIDDEN), lambda i: (0, 0, 0)),
            pl.BlockSpec((1, HIDDEN), lambda i: (0, 0)),
            pl.BlockSpec((HIDDEN, NUM_CLASSES), lambda i: (0, 0)),
            pl.BlockSpec((1, NUM_CLASSES), lambda i: (0, 0)),
        ],
        out_specs=pl.BlockSpec((_BB, NUM_CLASSES), lambda i: (i, 0)),
        out_shape=jax.ShapeDtypeStruct((BATCH, NUM_CLASSES), jnp.float32),
    )(e, W1c, b1, W2, b2)


def kernel(x, table, W1, b1, W2, b2):
    tabT = table.T
    tab_flat = _sc_detile(tabT, tabT[48:50, :], tabT[:, V_MAIN:])
    xr = x.T.astype(jnp.int32).reshape(-1)
    e = _sc_gather(xr, tab_flat)
    W1c = W1.reshape(CTX, EMBED, HIDDEN)
    return _mlp(e, W1c, b1.reshape(1, HIDDEN), W2, b2.reshape(1, NUM_CLASSES))


# speedup vs baseline: 6.8053x; 1.0005x over previous
import functools

import jax
import jax.numpy as jnp
from jax import lax
from jax.experimental import pallas as pl
from jax.experimental.pallas import tpu as pltpu
from jax.experimental.pallas import tpu_sc as plsc

VOCAB = 1000000
EMBED = 50
CTX = 5
HIDDEN = 256
NUM_CLASSES = 64
BATCH = 16384

NC = 2
NS = 16
NW = NC * NS

N_IDX = BATCH * CTX
PER_W = N_IDX // NW
CHUNK = 128
N_CHUNKS = PER_W // CHUNK
EPAD = 56

_sc_mesh = plsc.VectorSubcoreMesh(core_axis_name="c", subcore_axis_name="s")

VB = 2304
V_MAIN = (VOCAB // 128) * 128
NVB = V_MAIN // VB
TAIL = VOCAB - V_MAIN
NF8_FULL = EMBED // 8
N_MAIN = NF8_FULL * NVB
N_REST = NVB + 1
MAIN_PW = -(-N_MAIN // NW)
PAIRS_PW = -(-MAIN_PW // 2)
REST_PW = -(-N_REST // NW)


@functools.partial(
    pl.kernel,
    mesh=_sc_mesh,
    out_type=jax.ShapeDtypeStruct((EMBED * VOCAB,), jnp.float32),
    scratch_types=[
        pltpu.VMEM((8, VB), jnp.float32),
        pltpu.VMEM((8, VB), jnp.float32),
        pltpu.VMEM((8 * VB,), jnp.float32),
        pltpu.VMEM((2, VB), jnp.float32),
        pltpu.VMEM((EMBED, TAIL), jnp.float32),
        pltpu.VMEM((EMBED * TAIL,), jnp.float32),
        pltpu.SemaphoreType.DMA,
        pltpu.SemaphoreType.DMA,
        pltpu.SemaphoreType.DMA,
    ],
)
def _sc_detile(tabT_hbm, last2_hbm, tail_hbm, out_hbm,
               blk_a, blk_b, row_v, blk2_v, tail_v, trow_v,
               rsem_a, rsem_b, wsem):
    wid = lax.axis_index("s") * NC + lax.axis_index("c")

    def _src(unit):
        f8 = unit // NVB
        v0 = pl.multiple_of((unit % NVB) * VB, VB)
        return tabT_hbm.at[pl.ds(pl.multiple_of(f8 * 8, 8), 8),
                           pl.ds(v0, VB)], f8 * 8, v0

    def _compact_and_write(src_v, nrows, width, base_f, v0):
        for s in range(nrows):
            for t in range(width // 16):
                row_v[pl.ds(s * width + t * 16, 16)] = (
                    src_v[s, pl.ds(t * 16, 16)])
        copies = [
            pltpu.async_copy(
                row_v.at[pl.ds(s * width, width)],
                out_hbm.at[pl.ds((base_f + s) * VOCAB + v0, width)], wsem)
            for s in range(nrows)
        ]
        for cp in copies:
            cp.wait()

    @pl.loop(0, PAIRS_PW)
    def _pair(p):
        u_a = wid + (2 * p) * NW
        u_b = wid + (2 * p + 1) * NW

        @pl.when(u_a < N_MAIN)
        def _fire_a():
            src, _, _ = _src(u_a)
            pltpu.async_copy(src, blk_a, rsem_a)

        @pl.when(u_b < N_MAIN)
        def _fire_b():
            src, _, _ = _src(u_b)
            pltpu.async_copy(src, blk_b, rsem_b)

        @pl.when(u_a < N_MAIN)
        def _do_a():
            src, base_f, v0 = _src(u_a)
            pltpu.make_async_copy(src, blk_a, rsem_a).wait()
            _compact_and_write(blk_a, 8, VB, base_f, v0)

        @pl.when(u_b < N_MAIN)
        def _do_b():
            src, base_f, v0 = _src(u_b)
            pltpu.make_async_copy(src, blk_b, rsem_b).wait()
            _compact_and_write(blk_b, 8, VB, base_f, v0)

    @pl.loop(0, REST_PW)
    def _rest(i):
        unit = wid + i * NW

        @pl.when(unit < NVB)
        def _last2():
            v0 = pl.multiple_of(unit * VB, VB)
            pltpu.sync_copy(last2_hbm.at[:, pl.ds(v0, VB)], blk2_v)
            _compact_and_write(blk2_v, 2, VB, 48, v0)

        @pl.when(unit == NVB)
        def _tail():
            pltpu.sync_copy(tail_hbm, tail_v)
            for f in range(EMBED):
                for t in range(TAIL // 16):
                    trow_v[pl.ds(f * TAIL + t * 16, 16)] = (
                        tail_v[f, pl.ds(t * 16, 16)])
            copies = [
                pltpu.async_copy(
                    trow_v.at[pl.ds(f * TAIL, TAIL)],
                    out_hbm.at[pl.ds(f * VOCAB + V_MAIN, TAIL)], wsem)
                for f in range(EMBED)
            ]
            for cp in copies:
                cp.wait()


@functools.partial(
    pl.kernel,
    mesh=_sc_mesh,
    out_type=jax.ShapeDtypeStruct((CTX, EMBED, BATCH), jnp.float32),
    scratch_types=[
        pltpu.VMEM((CHUNK,), jnp.int32),
        pltpu.VMEM((CHUNK,), jnp.int32),
        pltpu.VMEM((EMBED, CHUNK), jnp.int32),
        pltpu.VMEM((EMBED, CHUNK), jnp.int32),
        pltpu.VMEM((EMBED, CHUNK), jnp.float32),
        pltpu.VMEM((EMBED, CHUNK), jnp.float32),
        pltpu.SemaphoreType.DMA,
        pltpu.SemaphoreType.DMA,
    ],
)
def _sc_gather(xr_hbm, tab_hbm, out_hbm,
               x_a, x_b, widx_a, widx_b, dst_a, dst_b, sem_a, sem_b):
    wid = lax.axis_index("s") * NC + lax.axis_index("c")

    def _stage_and_fire(chunk_id, x_v, widx_v, dst_v, sem):
        pltpu.sync_copy(xr_hbm.at[pl.ds(chunk_id * CHUNK, CHUNK)], x_v)
        for g in range(CHUNK // 16):
            rv = x_v[pl.ds(g * 16, 16)]
            for f in range(EMBED):
                widx_v[f, pl.ds(g * 16, 16)] = rv + (f * VOCAB)
        return [
            pltpu.async_copy(tab_hbm.at[widx_v.at[f]], dst_v.at[f], sem)
            for f in range(EMBED)
        ]

    def _drain_and_write(chunk_id, copies, dst_v):
        for cp in copies:
            cp.wait()
        c = chunk_id * CHUNK // BATCH
        b0 = chunk_id * CHUNK % BATCH
        pltpu.sync_copy(dst_v, out_hbm.at[c, :, pl.ds(b0, CHUNK)])

    @pl.loop(0, N_CHUNKS // 2)
    def _pair(p):
        u_a = wid * N_CHUNKS + 2 * p
        u_b = u_a + 1
        cp_a = _stage_and_fire(u_a, x_a, widx_a, dst_a, sem_a)
        cp_b = _stage_and_fire(u_b, x_b, widx_b, dst_b, sem_b)
        _drain_and_write(u_a, cp_a, dst_a)
        _drain_and_write(u_b, cp_b, dst_b)


_BB = 2048


def _mlp_body(e_ref, w1_ref, b1_ref, w2_ref, b2_ref, out_ref):
    acc = jnp.broadcast_to(b1_ref[...], (_BB, HIDDEN))
    for c in range(CTX):
        acc = acc + lax.dot_general(
            e_ref[c], w1_ref[c],
            dimension_numbers=(((0,), (0,)), ((), ())),
            preferred_element_type=jnp.float32,
            precision=lax.Precision.HIGHEST)
    h = jnp.tanh(acc)
    logits = jnp.dot(h, w2_ref[...],
                     preferred_element_type=jnp.float32,
                     precision=lax.Precision.HIGHEST) + b2_ref[...]
    m = jnp.max(logits, axis=1, keepdims=True)
    l = logits - m
    lse = jnp.log(jnp.sum(jnp.exp(l), axis=1, keepdims=True))
    out_ref[...] = l - lse


def _mlp(e, W1c, b1, W2, b2):
    grid = (BATCH // _BB,)
    return pl.pallas_call(
        _mlp_body,
        grid=grid,
        in_specs=[
            pl.BlockSpec((CTX, EMBED, _BB), lambda i: (0, 0, i)),
            pl.BlockSpec((CTX, EMBED, HIDDEN), lambda i: (0, 0, 0)),
            pl.BlockSpec((1, HIDDEN), lambda i: (0, 0)),
            pl.BlockSpec((HIDDEN, NUM_CLASSES), lambda i: (0, 0)),
            pl.BlockSpec((1, NUM_CLASSES), lambda i: (0, 0)),
        ],
        out_specs=pl.BlockSpec((_BB, NUM_CLASSES), lambda i: (i, 0)),
        out_shape=jax.ShapeDtypeStruct((BATCH, NUM_CLASSES), jnp.float32),
    )(e, W1c, b1, W2, b2)


def kernel(x, table, W1, b1, W2, b2):
    tabT = table.T
    tab_flat = _sc_detile(tabT, tabT[48:50, :], tabT[:, V_MAIN:])
    xr = x.T.astype(jnp.int32).reshape(-1)
    e = _sc_gather(xr, tab_flat)
    W1c = W1.reshape(CTX, EMBED, HIDDEN)
    return _mlp(e, W1c, b1.reshape(1, HIDDEN), W2, b2.reshape(1, NUM_CLASSES))
